# Initial kernel scaffold; baseline (speedup 1.0000x reference)
import jax
import jax.numpy as jnp
from jax.experimental import pallas as pl


def kernel(x, edge_index, edge_type, w1, root1, b1, w2, root2, b2):
    raise NotImplementedError("write your pallas kernel here")


# speedup vs baseline: 26.4429x; 26.4429x over previous
import functools

import jax
import jax.numpy as jnp
from jax import lax
from jax.experimental import pallas as pl
from jax.experimental.pallas import tpu as pltpu
from jax.experimental.pallas import tpu_sc as plsc

N = 10000
E = 320000
D_IN = 128
HID = 32
HH = 16
R = 5
NC = 200

NR = N * R
W = 128
EP = 327680
ROWS = EP // W
ROWS_PER_TILE = ROWS // 16
CHUNK_ROWS = 16
N_CHUNKS = ROWS_PER_TILE // CHUNK_ROWS
PADSEG = NR
ZC = 3128
ZC_LAST = NR - 15 * ZC

_f32 = jnp.float32


def _cooperative_span(s):
    return s * ZC


def _make_sc_agg():
    mesh = plsc.VectorSubcoreMesh(core_axis_name="c", subcore_axis_name="s")
    out_type = jax.ShapeDtypeStruct((2, NR, HH), _f32)
    scratch = [
        pltpu.VMEM((CHUNK_ROWS, W), jnp.int32),
        pltpu.VMEM((CHUNK_ROWS, W), jnp.int32),
        pltpu.VMEM((CHUNK_ROWS * W, HH), _f32),
        pltpu.VMEM_SHARED((NR + 16, HH), _f32),
        pltpu.SemaphoreType.DMA,
    ]

    @functools.partial(
        pl.kernel, mesh=mesh, out_type=out_type, scratch_types=scratch,
        compiler_params=pltpu.CompilerParams(use_tc_tiling_on_sc=False))
    def k(table, gidx, sidx, z2d, acc_out, gidx_v, sidx_v, rows_v, acc_sh,
          sem):
        c = lax.axis_index("c")
        s = lax.axis_index("s")
        zbase = s * ZC
        tab0 = table.at[0]
        tab1 = table.at[1]

        @pl.when(s < 15)
        def _():
            pltpu.sync_copy(z2d, acc_sh.at[pl.ds(zbase, ZC)])

        @pl.when(s == 15)
        def _():
            pltpu.sync_copy(z2d.at[pl.ds(0, ZC_LAST)],
                            acc_sh.at[pl.ds(15 * ZC, ZC_LAST)])

        plsc.subcore_barrier()

        row0 = s * ROWS_PER_TILE
        for chunk in range(N_CHUNKS):
            r0 = row0 + chunk * CHUNK_ROWS
            pltpu.sync_copy(gidx.at[pl.ds(r0, CHUNK_ROWS)], gidx_v)
            pltpu.sync_copy(sidx.at[pl.ds(r0, CHUNK_ROWS)], sidx_v)

            def _gather(tab):
                handles = []
                for j in range(CHUNK_ROWS):
                    handles.append(pltpu.async_copy(
                        tab.at[gidx_v.at[j]],
                        rows_v.at[pl.ds(j * W, W)], sem))
                for h in handles:
                    h.wait()

            @pl.when(c == 0)
            def _():
                _gather(tab0)

            @pl.when(c == 1)
            def _():
                _gather(tab1)

            for j in range(CHUNK_ROWS):
                pltpu.sync_copy(rows_v.at[pl.ds(j * W, W)],
                                acc_sh.at[sidx_v.at[j]], add=True)
        plsc.subcore_barrier()

        @pl.when(s < 15)
        def _():
            pltpu.sync_copy(acc_sh.at[pl.ds(zbase, ZC)],
                            acc_out.at[c, pl.ds(zbase, ZC)])

        @pl.when(s == 15)
        def _():
            pltpu.sync_copy(acc_sh.at[pl.ds(15 * ZC, ZC_LAST)],
                            acc_out.at[c, pl.ds(15 * ZC, ZC_LAST)])

    return k


def _make_sc_cnt():
    mesh = plsc.VectorSubcoreMesh(core_axis_name="c", subcore_axis_name="s")
    out_type = jax.ShapeDtypeStruct((NR, HH), _f32)
    scratch = [
        pltpu.VMEM((CHUNK_ROWS, W), jnp.int32),
        pltpu.VMEM((W, HH), _f32),
        pltpu.VMEM_SHARED((NR + 16, HH), _f32),
    ]

    @functools.partial(
        pl.kernel, mesh=mesh, out_type=out_type, scratch_types=scratch,
        compiler_params=pltpu.CompilerParams(use_tc_tiling_on_sc=False))
    def k(sidx, ones_h, z2d, cnt_out, sidx_v, ones_v, cnt_sh):
        c = lax.axis_index("c")
        s = lax.axis_index("s")
        zbase = s * ZC

        @pl.when(jnp.logical_and(c == 0, s < 15))
        def _():
            pltpu.sync_copy(z2d, cnt_sh.at[pl.ds(zbase, ZC)])

        @pl.when(jnp.logical_and(c == 0, s == 15))
        def _():
            pltpu.sync_copy(z2d.at[pl.ds(0, ZC_LAST)],
                            cnt_sh.at[pl.ds(15 * ZC, ZC_LAST)])

        @pl.when(c == 0)
        def _():
            pltpu.sync_copy(ones_h, ones_v)
        plsc.subcore_barrier()

        @pl.when(c == 0)
        def _():
            row0 = s * ROWS_PER_TILE
            for chunk in range(N_CHUNKS):
                r0 = row0 + chunk * CHUNK_ROWS
                pltpu.sync_copy(sidx.at[pl.ds(r0, CHUNK_ROWS)], sidx_v)
                for j in range(CHUNK_ROWS):
                    pltpu.sync_copy(ones_v, cnt_sh.at[sidx_v.at[j]],
                                    add=True)
        plsc.subcore_barrier()

        @pl.when(jnp.logical_and(c == 0, s < 15))
        def _():
            pltpu.sync_copy(cnt_sh.at[pl.ds(zbase, ZC)],
                            cnt_out.at[pl.ds(zbase, ZC)])

        @pl.when(jnp.logical_and(c == 0, s == 15))
        def _():
            pltpu.sync_copy(cnt_sh.at[pl.ds(15 * ZC, ZC_LAST)],
                            cnt_out.at[pl.ds(15 * ZC, ZC_LAST)])

    return k


_sc_agg = _make_sc_agg()
_sc_cnt = _make_sc_cnt()


_BN = 1000
_GRID = N // _BN


def _mm1_body(x_ref, w1_ref, tab_ref):
    xb = x_ref[...]
    for r in range(R):
        mr = jnp.dot(xb, w1_ref[r], preferred_element_type=_f32)
        tab_ref[0, r] = mr[:, :HH]
        tab_ref[1, r] = mr[:, HH:]


def _tc_mm1(x, w1):
    return pl.pallas_call(
        _mm1_body,
        grid=(_GRID,),
        in_specs=[
            pl.BlockSpec((_BN, D_IN), lambda i: (i, 0)),
            pl.BlockSpec((R, D_IN, HID), lambda i: (0, 0, 0)),
        ],
        out_specs=pl.BlockSpec((2, R, _BN, HH), lambda i: (0, 0, i, 0)),
        out_shape=jax.ShapeDtypeStruct((2, R, N, HH), _f32),
    )(x, w1)


def _combine1_body(acc_ref, cnt_ref, x_ref, root1_ref, b1_ref, root2_ref,
                   b2_ref, h1p_ref, hr_ref, recip_ref):
    recips = []
    for r in range(R):
        rec = 1.0 / jnp.maximum(cnt_ref[r, :, 0:1], 1.0)
        recips.append(rec)
        recip_ref[r] = rec
    xr = jnp.dot(x_ref[...], root1_ref[...], preferred_element_type=_f32)
    halves = []
    for h in range(2):
        sh = xr[:, h * HH:(h + 1) * HH] + b1_ref[:, h * HH:(h + 1) * HH]
        for r in range(R):
            sh = sh + acc_ref[h, r] * recips[r]
        hh = jnp.maximum(sh, 0.0)
        h1p_ref[h] = hh
        halves.append(hh)
    hr_ref[...] = (jnp.dot(halves[0], root2_ref[:HH],
                           preferred_element_type=_f32)
                   + jnp.dot(halves[1], root2_ref[HH:],
                             preferred_element_type=_f32)
                   + b2_ref[...])


def _tc_combine1(acc1, cnt, x, root1, b1, root2, b2):
    return pl.pallas_call(
        _combine1_body,
        grid=(_GRID,),
        in_specs=[
            pl.BlockSpec((2, R, _BN, HH), lambda i: (0, 0, i, 0)),
            pl.BlockSpec((R, _BN, HH), lambda i: (0, i, 0)),
            pl.BlockSpec((_BN, D_IN), lambda i: (i, 0)),
            pl.BlockSpec((D_IN, HID), lambda i: (0, 0)),
            pl.BlockSpec((1, HID), lambda i: (0, 0)),
            pl.BlockSpec((HID, NC), lambda i: (0, 0)),
            pl.BlockSpec((1, NC), lambda i: (0, 0)),
        ],
        out_specs=[
            pl.BlockSpec((2, _BN, HH), lambda i: (0, i, 0)),
            pl.BlockSpec((_BN, NC), lambda i: (i, 0)),
            pl.BlockSpec((R, _BN, 1), lambda i: (0, i, 0)),
        ],
        out_shape=[
            jax.ShapeDtypeStruct((2, N, HH), _f32),
            jax.ShapeDtypeStruct((N, NC), _f32),
            jax.ShapeDtypeStruct((R, N, 1), _f32),
        ],
    )(acc1, cnt, x, root1, b1, root2, b2)


def _combine2_body(acc_ref, recip_ref, hr_ref, w2_ref, out_ref):
    logits = hr_ref[...]
    for r in range(R):
        recip = recip_ref[r]
        for h in range(2):
            mean_h = acc_ref[h, r] * recip
            logits = logits + jnp.dot(
                mean_h, w2_ref[r, h * HH:(h + 1) * HH],
                preferred_element_type=_f32)
    m = jnp.max(logits, axis=-1, keepdims=True)
    shifted = logits - m
    lse = jnp.log(jnp.sum(jnp.exp(shifted), axis=-1, keepdims=True))
    out_ref[...] = shifted - lse


def _tc_combine2(acc2, recip, hr, w2):
    return pl.pallas_call(
        _combine2_body,
        grid=(_GRID,),
        in_specs=[
            pl.BlockSpec((2, R, _BN, HH), lambda i: (0, 0, i, 0)),
            pl.BlockSpec((R, _BN, 1), lambda i: (0, i, 0)),
            pl.BlockSpec((_BN, NC), lambda i: (i, 0)),
            pl.BlockSpec((R, HID, NC), lambda i: (0, 0, 0)),
        ],
        out_specs=pl.BlockSpec((_BN, NC), lambda i: (i, 0)),
        out_shape=jax.ShapeDtypeStruct((N, NC), _f32),
    )(acc2, recip, hr, w2)


def kernel(x, edge_index, edge_type, w1, root1, b1, w2, root2, b2):
    src = edge_index[0]
    dst = edge_index[1]
    et = edge_type

    npad = EP - E
    pad0 = jnp.zeros((npad,), jnp.int32)
    g1 = jnp.concatenate([et * N + src, pad0]).reshape(ROWS, W)
    g2 = jnp.concatenate([src, pad0]).reshape(ROWS, W)
    seg = jnp.concatenate([et * N + dst,
                           jnp.full((npad,), PADSEG, jnp.int32)]).reshape(ROWS, W)

    z2d = jnp.zeros((ZC, HH), _f32)
    ones2d = jnp.ones((W, HH), _f32)

    cnt = _sc_cnt(seg, ones2d, z2d)
    tab1 = _tc_mm1(x, w1)

    acc1 = _sc_agg(tab1.reshape(2, R * N, HH), g1, seg, z2d)
    acc1 = acc1.reshape(2, R, N, HH)
    cntv = cnt.reshape(R, N, HH)

    h1p, hr, recip = _tc_combine1(acc1, cntv, x, root1, b1.reshape(1, HID),
                                  root2, b2.reshape(1, NC))

    acc2 = _sc_agg(h1p, g2, seg, z2d)
    acc2 = acc2.reshape(2, R, N, HH)

    return _tc_combine2(acc2, recip, hr, w2)
